# baseline (speedup 1.0000x reference)
import functools

import jax
import jax.numpy as jnp
from jax import lax
from jax.experimental import pallas as pl
from jax.experimental.pallas import tpu as pltpu
from jax.experimental.pallas import tpu_sc as plsc

_TOPK = 5
_NEG = -jnp.inf


def _query_body(x_ref, q_ref):
    i = pl.program_id(0)
    x = x_ref[...]
    q = jnp.mean(x, axis=(2, 3))
    n = jnp.sqrt(jnp.sum(q * q, axis=1, keepdims=True))
    q_ref[pl.ds(i, 1), :] = q / jnp.maximum(n, 1e-12)


def _query_call(x):
    B, D, H, W = x.shape
    return pl.pallas_call(
        _query_body,
        grid=(B,),
        in_specs=[pl.BlockSpec((1, D, H, W), lambda i: (i, 0, 0, 0))],
        out_specs=pl.BlockSpec((B, D), lambda i: (0, 0)),
        out_shape=jax.ShapeDtypeStruct((B, D), jnp.float32),
    )(x)


def _topk_body(q_ref, k_ref, vals_ref, idx_ref, wb_ref, *, blk, nq):
    i = pl.program_id(0)

    @pl.when(i == 0)
    def _init():
        vals_ref[...] = jnp.full((nq, 16), _NEG, jnp.float32)
        idx_ref[...] = jnp.zeros((nq, 16), jnp.int32)

    k = k_ref[...]
    nsq = jnp.sum(k * k, axis=1, keepdims=True)
    kn = k / jnp.maximum(jnp.sqrt(nsq), 1e-12)
    q = q_ref[...]
    sim = lax.dot_general(q, kn, (((1,), (1,)), ((), ())),
                          preferred_element_type=jnp.float32)

    cidx = lax.broadcasted_iota(jnp.int32, (nq, blk), 1)
    big = jnp.int32(2 ** 31 - 1)
    base = i * blk
    new_v, new_i = [], []
    for _ in range(_TOPK):
        m = jnp.max(sim, axis=1, keepdims=True)
        p = jnp.min(jnp.where(sim == m, cidx, big), axis=1, keepdims=True)
        new_v.append(m)
        new_i.append(p + base)
        sim = jnp.where(cidx == p, _NEG, sim)

    rv = vals_ref[...]
    ri = idx_ref[...]
    candv = jnp.concatenate(
        [rv[:, :_TOPK]] + new_v + [jnp.full((nq, 6), _NEG, jnp.float32)], axis=1)
    candi = jnp.concatenate(
        [ri[:, :_TOPK]] + new_i + [jnp.zeros((nq, 6), jnp.int32)], axis=1)
    c16 = lax.broadcasted_iota(jnp.int32, (nq, 16), 1)
    out_v, out_i = [], []
    for _ in range(_TOPK):
        m = jnp.max(candv, axis=1, keepdims=True)
        p = jnp.min(jnp.where(candv == m, c16, big), axis=1, keepdims=True)
        sel = jnp.sum(jnp.where(c16 == p, candi, 0), axis=1, keepdims=True)
        out_v.append(m)
        out_i.append(sel)
        candv = jnp.where(c16 == p, _NEG, candv)

    vals_ref[...] = jnp.concatenate(
        out_v + [jnp.full((nq, 11), _NEG, jnp.float32)], axis=1)
    idx_ref[...] = jnp.concatenate(
        out_i + [jnp.zeros((nq, 11), jnp.int32)], axis=1)

    @pl.when(i == pl.num_programs(0) - 1)
    def _softmax():
        w5 = jnp.concatenate(out_v, axis=1)
        mx = jnp.max(w5, axis=1, keepdims=True)
        e = jnp.exp(w5 - mx)
        w = e / jnp.sum(e, axis=1, keepdims=True)
        wb_ref[...] = jnp.concatenate(
            [jnp.broadcast_to(w[:, t:t + 1], (nq, 16)) for t in range(_TOPK)],
            axis=1)


def _topk_call(qn, keys):
    nq, D = qn.shape
    N = keys.shape[0]
    blk = 8000 if N % 8000 == 0 else N
    nsteps = N // blk
    body = functools.partial(_topk_body, blk=blk, nq=nq)
    return pl.pallas_call(
        body,
        grid=(nsteps,),
        in_specs=[pl.BlockSpec((nq, D), lambda i: (0, 0)),
                  pl.BlockSpec((blk, D), lambda i: (i, 0))],
        out_specs=[pl.BlockSpec((nq, 16), lambda i: (0, 0)),
                   pl.BlockSpec((nq, 16), lambda i: (0, 0)),
                   pl.BlockSpec((nq, 80), lambda i: (0, 0))],
        out_shape=[jax.ShapeDtypeStruct((nq, 16), jnp.float32),
                   jax.ShapeDtypeStruct((nq, 16), jnp.int32),
                   jax.ShapeDtypeStruct((nq, 80), jnp.float32)],
    )(qn, keys)


def _combine_sc(wbc, top_idx, values):
    nq = top_idx.shape[0]
    D = values.shape[1]
    mesh = plsc.VectorSubcoreMesh(core_axis_name="c", subcore_axis_name="s")

    @functools.partial(
        pl.kernel,
        mesh=mesh,
        out_type=jax.ShapeDtypeStruct((nq, D), jnp.float32),
        compiler_params=pltpu.CompilerParams(use_tc_tiling_on_sc=False),
        scratch_types=[
            pltpu.VMEM((80,), jnp.float32),
            pltpu.VMEM((16,), jnp.int32),
            pltpu.VMEM((16, D), jnp.float32),
            pltpu.VMEM((D,), jnp.float32),
            pltpu.SemaphoreType.DMA,
        ],
    )
    def _k(wb_hbm, ti_hbm, val_hbm, out_hbm, wb_v, ti_v, rows_v, ob_v, sem):
        wid = lax.axis_index("s") * 2 + lax.axis_index("c")
        pltpu.sync_copy(wb_hbm.at[wid], wb_v)
        pltpu.sync_copy(ti_hbm.at[wid], ti_v)
        pltpu.async_copy(val_hbm.at[ti_v], rows_v, sem).wait()

        ws = [wb_v[pl.ds(t * 16, 16)] for t in range(_TOPK)]
        for dc in range(D // 16):
            acc = rows_v[0, pl.ds(dc * 16, 16)] * ws[0]
            for t in range(1, _TOPK):
                acc = acc + rows_v[t, pl.ds(dc * 16, 16)] * ws[t]
            ob_v[pl.ds(dc * 16, 16)] = acc
        pltpu.sync_copy(ob_v, out_hbm.at[wid])

    return _k(wbc, top_idx, values)


def _bcast_body(m_ref, o_ref, *, D, H, W):
    i = pl.program_id(0)
    row = m_ref[pl.ds(i, 1), :]
    o_ref[...] = jnp.broadcast_to(row[:, :, None, None], (1, D, H, W))


def _bcast_call(matched, H, W):
    B, D = matched.shape
    body = functools.partial(_bcast_body, D=D, H=H, W=W)
    return pl.pallas_call(
        body,
        grid=(B,),
        in_specs=[pl.BlockSpec((B, D), lambda i: (0, 0))],
        out_specs=pl.BlockSpec((1, D, H, W), lambda i: (i, 0, 0, 0)),
        out_shape=jax.ShapeDtypeStruct((B, D, H, W), jnp.float32),
    )(matched)


def kernel(x, keys, values):
    B, C, H, W = x.shape
    qn = _query_call(x)
    tv, ti, wbc = _topk_call(qn, keys)
    del tv
    matched = _combine_sc(wbc, ti, values)
    return _bcast_call(matched, H, W)


# speedup vs baseline: 1.1466x; 1.1466x over previous
import functools

import jax
import jax.numpy as jnp
from jax import lax
from jax.experimental import pallas as pl
from jax.experimental.pallas import tpu as pltpu
from jax.experimental.pallas import tpu_sc as plsc

_TOPK = 5
_NEG = -jnp.inf


def _query_body(x_ref, q_ref):
    i = pl.program_id(0)
    x = x_ref[...]
    q = jnp.mean(x, axis=(2, 3))
    n = jnp.sqrt(jnp.sum(q * q, axis=1, keepdims=True))
    q_ref[pl.ds(i, 1), :] = q / jnp.maximum(n, 1e-12)


def _query_call(x):
    B, D, H, W = x.shape
    return pl.pallas_call(
        _query_body,
        grid=(B,),
        in_specs=[pl.BlockSpec((1, D, H, W), lambda i: (i, 0, 0, 0))],
        out_specs=pl.BlockSpec((B, D), lambda i: (0, 0)),
        out_shape=jax.ShapeDtypeStruct((B, D), jnp.float32),
    )(x)


def _topk_body(q_ref, k_ref, vals_ref, idx_ref, wb_ref, *, blk, nq):
    i = pl.program_id(0)

    @pl.when(i == 0)
    def _init():
        vals_ref[...] = jnp.full((nq, 16), _NEG, jnp.float32)
        idx_ref[...] = jnp.zeros((nq, 16), jnp.int32)

    k = k_ref[...]
    q = q_ref[...]
    sim = lax.dot_general(q, k, (((1,), (1,)), ((), ())),
                          preferred_element_type=jnp.float32)
    ones8 = jnp.ones((8, k.shape[1]), jnp.float32)
    nsq = lax.dot_general(ones8, k * k, (((1,), (1,)), ((), ())),
                          preferred_element_type=jnp.float32)
    sim = sim / jnp.maximum(jnp.sqrt(nsq[0:1, :]), 1e-12)

    cidx = lax.broadcasted_iota(jnp.int32, (nq, blk), 1)
    big = jnp.int32(2 ** 31 - 1)
    base = i * blk
    new_v, new_i = [], []
    for _ in range(_TOPK):
        m = jnp.max(sim, axis=1, keepdims=True)
        p = jnp.min(jnp.where(sim == m, cidx, big), axis=1, keepdims=True)
        new_v.append(m)
        new_i.append(p + base)
        sim = jnp.where(cidx == p, _NEG, sim)

    rv = vals_ref[...]
    ri = idx_ref[...]
    candv = jnp.concatenate(
        [rv[:, :_TOPK]] + new_v + [jnp.full((nq, 6), _NEG, jnp.float32)], axis=1)
    candi = jnp.concatenate(
        [ri[:, :_TOPK]] + new_i + [jnp.zeros((nq, 6), jnp.int32)], axis=1)
    c16 = lax.broadcasted_iota(jnp.int32, (nq, 16), 1)
    out_v, out_i = [], []
    for _ in range(_TOPK):
        m = jnp.max(candv, axis=1, keepdims=True)
        p = jnp.min(jnp.where(candv == m, c16, big), axis=1, keepdims=True)
        sel = jnp.sum(jnp.where(c16 == p, candi, 0), axis=1, keepdims=True)
        out_v.append(m)
        out_i.append(sel)
        candv = jnp.where(c16 == p, _NEG, candv)

    vals_ref[...] = jnp.concatenate(
        out_v + [jnp.full((nq, 11), _NEG, jnp.float32)], axis=1)
    idx_ref[...] = jnp.concatenate(
        out_i + [jnp.zeros((nq, 11), jnp.int32)], axis=1)

    @pl.when(i == pl.num_programs(0) - 1)
    def _softmax():
        w5 = jnp.concatenate(out_v, axis=1)
        mx = jnp.max(w5, axis=1, keepdims=True)
        e = jnp.exp(w5 - mx)
        w = e / jnp.sum(e, axis=1, keepdims=True)
        wb_ref[...] = jnp.concatenate(
            [jnp.broadcast_to(w[:, t:t + 1], (nq, 16)) for t in range(_TOPK)],
            axis=1)


def _topk_call(qn, keys):
    nq, D = qn.shape
    N = keys.shape[0]
    blk = 20000 if N % 20000 == 0 else N
    nsteps = N // blk
    body = functools.partial(_topk_body, blk=blk, nq=nq)
    return pl.pallas_call(
        body,
        grid=(nsteps,),
        in_specs=[pl.BlockSpec((nq, D), lambda i: (0, 0)),
                  pl.BlockSpec((blk, D), lambda i: (i, 0))],
        out_specs=[pl.BlockSpec((nq, 16), lambda i: (0, 0)),
                   pl.BlockSpec((nq, 16), lambda i: (0, 0)),
                   pl.BlockSpec((nq, 80), lambda i: (0, 0))],
        out_shape=[jax.ShapeDtypeStruct((nq, 16), jnp.float32),
                   jax.ShapeDtypeStruct((nq, 16), jnp.int32),
                   jax.ShapeDtypeStruct((nq, 80), jnp.float32)],
    )(qn, keys)


def _combine_sc(wbc, top_idx, values):
    nq = top_idx.shape[0]
    D = values.shape[1]
    mesh = plsc.VectorSubcoreMesh(core_axis_name="c", subcore_axis_name="s")

    @functools.partial(
        pl.kernel,
        mesh=mesh,
        out_type=jax.ShapeDtypeStruct((nq, D), jnp.float32),
        compiler_params=pltpu.CompilerParams(use_tc_tiling_on_sc=False),
        scratch_types=[
            pltpu.VMEM((80,), jnp.float32),
            pltpu.VMEM((16,), jnp.int32),
            pltpu.VMEM((16, D), jnp.float32),
            pltpu.VMEM((D,), jnp.float32),
            pltpu.SemaphoreType.DMA,
        ],
    )
    def _k(wb_hbm, ti_hbm, val_hbm, out_hbm, wb_v, ti_v, rows_v, ob_v, sem):
        wid = lax.axis_index("s") * 2 + lax.axis_index("c")
        pltpu.sync_copy(wb_hbm.at[wid], wb_v)
        pltpu.sync_copy(ti_hbm.at[wid], ti_v)
        pltpu.async_copy(val_hbm.at[ti_v], rows_v, sem).wait()

        ws = [wb_v[pl.ds(t * 16, 16)] for t in range(_TOPK)]
        for dc in range(D // 16):
            acc = rows_v[0, pl.ds(dc * 16, 16)] * ws[0]
            for t in range(1, _TOPK):
                acc = acc + rows_v[t, pl.ds(dc * 16, 16)] * ws[t]
            ob_v[pl.ds(dc * 16, 16)] = acc
        pltpu.sync_copy(ob_v, out_hbm.at[wid])

    return _k(wbc, top_idx, values)


def _bcast_body(m_ref, o_ref, *, D, H, W):
    i = pl.program_id(0)
    row = m_ref[pl.ds(i, 1), :]
    o_ref[...] = jnp.broadcast_to(row[:, :, None, None], (1, D, H, W))


def _bcast_call(matched, H, W):
    B, D = matched.shape
    body = functools.partial(_bcast_body, D=D, H=H, W=W)
    return pl.pallas_call(
        body,
        grid=(B,),
        in_specs=[pl.BlockSpec((B, D), lambda i: (0, 0))],
        out_specs=pl.BlockSpec((1, D, H, W), lambda i: (i, 0, 0, 0)),
        out_shape=jax.ShapeDtypeStruct((B, D, H, W), jnp.float32),
    )(matched)


def kernel(x, keys, values):
    B, C, H, W = x.shape
    qn = _query_call(x)
    tv, ti, wbc = _topk_call(qn, keys)
    del tv
    matched = _combine_sc(wbc, ti, values)
    return _bcast_call(matched, H, W)
